# baseline (speedup 1.0000x reference)
import functools

import jax
import jax.numpy as jnp
import numpy as np
from jax import lax
from jax.experimental import pallas as pl
from jax.experimental.pallas import tpu as pltpu
from jax.experimental.pallas import tpu_sc as plsc

_BS = 4096
_NOBJ = 21
_P = 26
_E = 32
_ROWS = _BS * _NOBJ
_NIDX = _ROWS * _P
_NBAND = _ROWS // 8
_LT = 7
_SLOTS = _LT * 32

_NC = 2
_NS = 16
_NW = _NC * _NS
_BPW = _NBAND // _NW
_NB = 6
_NCHUNK = _BPW // _NB
_CIN = _NB * _P * 8
_COUT = _NB * _SLOTS

assert _BPW * _NW == _NBAND and _NCHUNK * _NB == _BPW
assert _CIN % 8 == 0 and _COUT % 8 == 0 and _NCHUNK % 2 == 0


def _slot_patterns():
    t = np.arange(_SLOTS)
    lt = t // 32
    s = (t % 32) // 4
    p = 4 * lt + t % 4
    valid = p < _P
    patt = np.where(valid, s * _P + p, 0).astype(np.int32)
    padm = (~valid).astype(np.int32)
    return patt, padm


_PATT_NP, _PADM_NP = _slot_patterns()


@functools.partial(
    pl.kernel,
    out_type=jax.ShapeDtypeStruct((_NBAND * _SLOTS, _E), jnp.float32),
    mesh=plsc.VectorSubcoreMesh(core_axis_name="c", subcore_axis_name="s"),
    scratch_types=[
        pltpu.VMEM((2, _CIN), jnp.int32),
        pltpu.VMEM((2, _COUT), jnp.int32),
        pltpu.VMEM((2, _COUT, _E), jnp.float32),
        pltpu.VMEM((_SLOTS,), jnp.int32),
        pltpu.VMEM((_SLOTS,), jnp.int32),
        pltpu.SemaphoreType.DMA,
        pltpu.SemaphoreType.DMA,
        pltpu.SemaphoreType.DMA,
        pltpu.SemaphoreType.DMA,
    ],
    compiler_params=pltpu.CompilerParams(
        use_tc_tiling_on_sc=False, needs_layout_passes=False
    ),
)
def _sc_gather(idx_hbm, table_hbm, patt_hbm, padm_hbm, out_hbm,
               idx_in, idx_out, rows_v, patt_v, padm_v, g0, g1, s0, s1):
    wid = lax.axis_index("s") * _NC + lax.axis_index("c")
    ibase = wid * _BPW * _P * 8
    obase = wid * _BPW * _SLOTS
    gsem = (g0, g1)
    ssem = (s0, s1)

    pltpu.sync_copy(patt_hbm, patt_v)
    pltpu.sync_copy(padm_hbm, padm_v)

    def gather_start(i, b):
        pltpu.sync_copy(idx_hbm.at[pl.ds(ibase + i * _CIN, _CIN)], idx_in.at[b])
        for k in range(_NB):
            for g in range(_SLOTS // 16):
                t0 = g * 16
                src = patt_v[pl.ds(t0, 16)] + k * (_P * 8)
                v = plsc.load_gather(idx_in.at[b], [src])
                pad = padm_v[pl.ds(t0, 16)] != 0
                spread = lax.iota(jnp.int32, 16) + t0
                idx_out[b, pl.ds(k * _SLOTS + t0, 16)] = jnp.where(pad, spread, v)
        pltpu.async_copy(table_hbm.at[idx_out.at[b]], rows_v.at[b], gsem[b])

    def gather_wait(b):
        pltpu.make_async_copy(table_hbm.at[idx_out.at[b]], rows_v.at[b], gsem[b]).wait()

    def store_start(i, b):
        pltpu.async_copy(rows_v.at[b], out_hbm.at[pl.ds(obase + i * _COUT, _COUT)], ssem[b])

    def store_wait(i, b):
        pltpu.make_async_copy(rows_v.at[b], out_hbm.at[pl.ds(obase + i * _COUT, _COUT)], ssem[b]).wait()

    gather_start(0, 0)
    gather_start(1, 1)

    def pair(j, carry):
        for b in range(2):
            i = 2 * j + b
            gather_wait(b)
            store_start(i, b)
            store_wait(i, b)
            gather_start(i + 2, b)
        return carry

    lax.fori_loop(0, (_NCHUNK - 2) // 2, pair, 0)

    for b in range(2):
        gather_wait(b)
        store_start(_NCHUNK - 2 + b, b)
    for b in range(2):
        store_wait(_NCHUNK - 2 + b, b)


_BB = 128
_RB = _BB * 8


def _tc_proj(g_ref, xs_ref, w_ref, b_ref, ma_ref, y_ref, m_ref):
    y = jnp.dot(g_ref[:, 0].reshape(_RB, 128), w_ref[0],
                preferred_element_type=jnp.float32)
    for lt in range(1, _LT):
        y = y + jnp.dot(g_ref[:, lt].reshape(_RB, 128), w_ref[lt],
                        preferred_element_type=jnp.float32)
    y = y + b_ref[...]
    pad = jnp.sum(xs_ref[...], axis=1, keepdims=True) == 0
    y_ref[...] = jnp.where(pad, ma_ref[...], y)
    m_ref[...] = pad.astype(jnp.int32)


def kernel(x, table, W, b, mark_absent):
    idx_flat = x.reshape(_NIDX)
    patt = jnp.asarray(_PATT_NP)
    padm = jnp.asarray(_PADM_NP)
    gathered = _sc_gather(idx_flat, table, patt, padm)

    g4 = gathered.reshape(_NBAND, _LT, 8, 128)
    xs = x.reshape(_ROWS, _P)
    w4 = jnp.pad(W.T, ((0, _LT * 128 - _P * _E), (0, 0))).reshape(_LT, 128, _E)
    y, m = pl.pallas_call(
        _tc_proj,
        grid=(_NBAND // _BB,),
        in_specs=[
            pl.BlockSpec((_BB, _LT, 8, 128), lambda i: (i, 0, 0, 0)),
            pl.BlockSpec((_RB, _P), lambda i: (i, 0)),
            pl.BlockSpec((_LT, 128, _E), lambda i: (0, 0, 0)),
            pl.BlockSpec((1, _E), lambda i: (0, 0)),
            pl.BlockSpec((1, _E), lambda i: (0, 0)),
        ],
        out_specs=[
            pl.BlockSpec((_RB, _E), lambda i: (i, 0)),
            pl.BlockSpec((_RB, 1), lambda i: (i, 0)),
        ],
        out_shape=[
            jax.ShapeDtypeStruct((_ROWS, _E), jnp.float32),
            jax.ShapeDtypeStruct((_ROWS, 1), jnp.int32),
        ],
    )(g4, xs, w4, b.reshape(1, _E), mark_absent.reshape(1, _E))

    obj_emb = y.reshape(_BS, _NOBJ, _E)
    padding = m.reshape(_BS, _NOBJ) != 0
    return obj_emb, padding
